# baseline (speedup 1.0000x reference)
import functools

import jax
import jax.numpy as jnp
from jax import lax
from jax.experimental import pallas as pl
from jax.experimental.pallas import tpu as pltpu
from jax.experimental.pallas import tpu_sc as plsc

NC = 2
NS = 16
NW = NC * NS

CH = 80


def _deg_body(dst_hbm, out_hbm, idx_v, ones_v, zero_v, deg_sh, sem):
  n = out_hbm.shape[0] // NC
  e = dst_hbm.shape[0]
  cid = lax.axis_index("c")
  sid = lax.axis_index("s")
  wid = sid * NC + cid
  epw = e // NW
  nch = epw // CH

  for j in range(CH // 16):
    ones_v[pl.ds(j * 16, 16)] = jnp.full((16,), 1.0, jnp.float32)

  @pl.when(sid == 0)
  def _():
    def zfill(j, _):
      zero_v[pl.ds(j * 16, 16)] = jnp.zeros((16,), jnp.float32)
      return 0
    lax.fori_loop(0, n // 16, zfill, 0)
    pltpu.sync_copy(zero_v, deg_sh)

  plsc.subcore_barrier()

  def chunk(c, _):
    base = wid * epw + c * CH
    pltpu.sync_copy(dst_hbm.at[pl.ds(base, CH)], idx_v)
    pltpu.sync_copy(ones_v, deg_sh.at[idx_v], add=True)
    return 0

  lax.fori_loop(0, nch, chunk, 0)
  plsc.subcore_barrier()

  @pl.when(sid == 0)
  def _():
    pltpu.sync_copy(deg_sh, zero_v)
    pltpu.sync_copy(zero_v, out_hbm.at[pl.ds(cid * n, n)])


def _sc_degree(dst):
  e = dst.shape[0]
  n = 10000
  mesh = plsc.VectorSubcoreMesh(core_axis_name="c", subcore_axis_name="s", num_cores=NC, num_subcores=NS)
  return pl.kernel(
      _deg_body,
      out_type=jax.ShapeDtypeStruct((NC * n,), jnp.float32),
      mesh=mesh,
      scratch_types=[
          pltpu.VMEM((CH,), jnp.int32),
          pltpu.VMEM((CH,), jnp.float32),
          pltpu.VMEM((n,), jnp.float32),
          pltpu.VMEM_SHARED((n,), jnp.float32),
          pltpu.SemaphoreType.DMA,
      ],
  )(dst)


def _scatter_body(y_hbm, src_hbm, dst_hbm, out_hbm, is0, is1, id0, id1, zi0,
                  zi1, rows0, rows1, zero_v, z_sh, sis0, sis1, sid0, sid1,
                  sg0, sg1, sz0, sz1):
  n, d = y_hbm.shape
  ncg, _, chl = src_hbm.shape
  cid = lax.axis_index("c")
  sid = lax.axis_index("s")
  wid = sid * NC + cid
  nch = (ncg // NW) + jnp.where(wid < (ncg % NW), 1, 0)

  zrows = zero_v.shape[0]
  nzg = n // zrows
  nz = (nzg // NS) + jnp.where(sid < (nzg % NS), 1, 0)

  def zfill(j, _):
    for k in range(d // 16):
      zero_v[j, pl.ds(k * 16, 16)] = jnp.zeros((16,), jnp.float32)
    return 0
  lax.fori_loop(0, zrows, zfill, 0)

  def zchunk(k, _):
    g = sid + NS * k
    pltpu.sync_copy(zero_v, z_sh.at[pl.ds(g * zrows, zrows)])
    return 0
  lax.fori_loop(0, nz, zchunk, 0)

  plsc.subcore_barrier()

  idx_s = [is0, is1]
  idx_d = [id0, id1]
  zidx = [zi0, zi1]
  rows = [rows0, rows1]
  sem_is = [sis0, sis1]
  sem_id = [sid0, sid1]
  sem_g = [sg0, sg1]
  sem_z = [sz0, sz1]

  def load_idx(c, s):
    g = wid + NW * c
    pltpu.async_copy(src_hbm.at[g, 0], idx_s[s], sem_is[s])
    pltpu.async_copy(dst_hbm.at[g, 0], idx_d[s], sem_id[s])

  def wait_idx(c, s):
    g = wid + NW * c
    pltpu.make_async_copy(src_hbm.at[g, 0], idx_s[s], sem_is[s]).wait()
    pltpu.make_async_copy(dst_hbm.at[g, 0], idx_d[s], sem_id[s]).wait()

  def wait_scatter(s):
    pltpu.make_async_copy(rows[s], z_sh.at[zidx[s]], sem_z[s]).wait()

  load_idx(0, 0)
  wait_idx(0, 0)
  pltpu.async_copy(y_hbm.at[idx_s[0]], rows[0], sem_g[0])
  load_idx(1, 1)

  def chunk(c, _):
    for s in range(2):
      @pl.when(c % 2 == s)
      def _():
        t = 1 - s

        @pl.when(c >= 1)
        def _():
          wait_scatter(t)

        @pl.when(c + 1 < nch)
        def _():
          wait_idx(c + 1, t)
          pltpu.async_copy(y_hbm.at[idx_s[t]], rows[t], sem_g[t])

        pltpu.make_async_copy(y_hbm.at[idx_s[s]], rows[s], sem_g[s]).wait()
        for j in range(chl // 16):
          sl = pl.ds(j * 16, 16)
          zidx[s][sl] = idx_d[s][sl]
        pltpu.async_copy(rows[s], z_sh.at[zidx[s]], sem_z[s], add=True)

        @pl.when(c + 2 < nch)
        def _():
          load_idx(c + 2, s)
    return 0

  lax.fori_loop(0, nch, chunk, 0)

  for s in range(2):
    @pl.when((nch - 1) % 2 == s)
    def _():
      wait_scatter(s)

  plsc.subcore_barrier()

  nog = n // 80
  no = (nog // NS) + jnp.where(sid < (nog % NS), 1, 0)

  def ochunk(k, _):
    g = sid + NS * k
    r0 = g * 80
    pltpu.sync_copy(z_sh.at[pl.ds(r0, 80)], out_hbm.at[cid, pl.ds(r0, 80)])
    return 0
  lax.fori_loop(0, no, ochunk, 0)


def _sc_node_scatter(y, src2, dst2):
  n, d = y.shape
  chl = src2.shape[2]
  mesh = plsc.VectorSubcoreMesh(core_axis_name="c", subcore_axis_name="s", num_cores=NC, num_subcores=NS)
  return pl.kernel(
      _scatter_body,
      out_type=jax.ShapeDtypeStruct((NC, n, d), jnp.float32),
      mesh=mesh,
      scratch_types=[
          pltpu.VMEM((chl,), jnp.int32),
          pltpu.VMEM((chl,), jnp.int32),
          pltpu.VMEM((chl,), jnp.int32),
          pltpu.VMEM((chl,), jnp.int32),
          pltpu.VMEM((chl,), jnp.int32),
          pltpu.VMEM((chl,), jnp.int32),
          pltpu.VMEM((chl, d), jnp.float32),
          pltpu.VMEM((chl, d), jnp.float32),
          pltpu.VMEM((40, d), jnp.float32),
          pltpu.VMEM_SHARED((n, d), jnp.float32),
          pltpu.SemaphoreType.DMA,
          pltpu.SemaphoreType.DMA,
          pltpu.SemaphoreType.DMA,
          pltpu.SemaphoreType.DMA,
          pltpu.SemaphoreType.DMA,
          pltpu.SemaphoreType.DMA,
          pltpu.SemaphoreType.DMA,
          pltpu.SemaphoreType.DMA,
      ],
  )(y, src2, dst2)


def _edge_body(a_hbm, b_hbm, src_hbm, dst_hbm, out_hbm, is0, is1, id0, id1,
               buf_a0, buf_a1, buf_b0, buf_b1, sis0, sis1, sid0, sid1,
               sem_a0, sem_a1, sem_b0, sem_b1, sem_o0, sem_o1):
  e, d = out_hbm.shape
  ncg, _, chl = src_hbm.shape
  cid = lax.axis_index("c")
  sid = lax.axis_index("s")
  wid = sid * NC + cid
  nch = (ncg // NW) + jnp.where(wid < (ncg % NW), 1, 0)

  idx_s = [is0, is1]
  idx_d = [id0, id1]
  buf_a = [buf_a0, buf_a1]
  buf_b = [buf_b0, buf_b1]
  sem_is = [sis0, sis1]
  sem_id = [sid0, sid1]
  sem_a = [sem_a0, sem_a1]
  sem_b = [sem_b0, sem_b1]
  sem_o = [sem_o0, sem_o1]

  def load_idx(c, s):
    g = wid + NW * c
    pltpu.async_copy(src_hbm.at[g, 0], idx_s[s], sem_is[s])
    pltpu.async_copy(dst_hbm.at[g, 0], idx_d[s], sem_id[s])

  def wait_idx(c, s):
    g = wid + NW * c
    pltpu.make_async_copy(src_hbm.at[g, 0], idx_s[s], sem_is[s]).wait()
    pltpu.make_async_copy(dst_hbm.at[g, 0], idx_d[s], sem_id[s]).wait()

  def issue_gathers(s):
    pltpu.async_copy(a_hbm.at[idx_s[s]], buf_a[s], sem_a[s])
    pltpu.async_copy(b_hbm.at[idx_d[s]], buf_b[s], sem_b[s])

  def wait_gathers(s):
    pltpu.make_async_copy(a_hbm.at[idx_s[s]], buf_a[s], sem_a[s]).wait()
    pltpu.make_async_copy(b_hbm.at[idx_d[s]], buf_b[s], sem_b[s]).wait()

  def wait_store(c, s):
    base = (wid + NW * c) * chl
    pltpu.make_async_copy(buf_a[s], out_hbm.at[pl.ds(base, chl)],
                          sem_o[s]).wait()

  load_idx(0, 0)
  wait_idx(0, 0)
  issue_gathers(0)
  load_idx(1, 1)

  def chunk(c, _):
    for s in range(2):
      @pl.when(c % 2 == s)
      def _():
        t = 1 - s

        @pl.when(c + 1 < nch)
        def _():
          wait_idx(c + 1, t)

          @pl.when(c >= 1)
          def _():
            wait_store(c - 1, t)
          issue_gathers(t)

        wait_gathers(s)

        @plsc.parallel_loop(0, chl, 1, unroll=4)
        def add_row(r):
          for j in range(d // 16):
            sl = pl.ds(j * 16, 16)
            buf_a[s][r, sl] = buf_a[s][r, sl] + buf_b[s][r, sl]

        base = (wid + NW * c) * chl
        pltpu.async_copy(buf_a[s], out_hbm.at[pl.ds(base, chl)], sem_o[s])

        @pl.when(c + 2 < nch)
        def _():
          load_idx(c + 2, s)
    return 0

  lax.fori_loop(0, nch, chunk, 0)

  for s in range(2):
    @pl.when((nch - 1) % 2 == s)
    def _():
      wait_store(nch - 2, 1 - s)
      wait_store(nch - 1, s)


def _sc_edge_combine(a, b, src2, dst2):
  e, d = a.shape
  chl = src2.shape[2]
  mesh = plsc.VectorSubcoreMesh(core_axis_name="c", subcore_axis_name="s", num_cores=NC, num_subcores=NS)
  return pl.kernel(
      _edge_body,
      out_type=jax.ShapeDtypeStruct((e, d), jnp.float32),
      mesh=mesh,
      scratch_types=[
          pltpu.VMEM((chl,), jnp.int32),
          pltpu.VMEM((chl,), jnp.int32),
          pltpu.VMEM((chl,), jnp.int32),
          pltpu.VMEM((chl,), jnp.int32),
          pltpu.VMEM((chl, d), jnp.float32),
          pltpu.VMEM((chl, d), jnp.float32),
          pltpu.VMEM((chl, d), jnp.float32),
          pltpu.VMEM((chl, d), jnp.float32),
          pltpu.SemaphoreType.DMA,
          pltpu.SemaphoreType.DMA,
          pltpu.SemaphoreType.DMA,
          pltpu.SemaphoreType.DMA,
          pltpu.SemaphoreType.DMA,
          pltpu.SemaphoreType.DMA,
          pltpu.SemaphoreType.DMA,
          pltpu.SemaphoreType.DMA,
          pltpu.SemaphoreType.DMA,
          pltpu.SemaphoreType.DMA,
      ],
  )(a, b, src2, dst2)


def _dinv_body(degp_ref, out_ref):
  deg = degp_ref[0, :] + degp_ref[1, :] + 1.0
  out_ref[0, :] = lax.rsqrt(deg)


def _tc_dinv(degp):
  return pl.pallas_call(
      _dinv_body,
      out_shape=jax.ShapeDtypeStruct((1, degp.shape[1]), jnp.float32),
  )(degp)


def _prep_body(x_ref, w_ref, dinv_ref, y_ref):
  xw = jnp.dot(x_ref[...], w_ref[...], preferred_element_type=jnp.float32)
  y_ref[...] = xw * dinv_ref[...]


def _tc_prep(x, w, dinv_col):
  n, d = x.shape
  blk = 1000
  return pl.pallas_call(
      _prep_body,
      grid=(n // blk,),
      in_specs=[
          pl.BlockSpec((blk, d), lambda i: (i, 0)),
          pl.BlockSpec((d, d), lambda i: (0, 0)),
          pl.BlockSpec((blk, 1), lambda i: (i, 0)),
      ],
      out_specs=pl.BlockSpec((blk, d), lambda i: (i, 0)),
      out_shape=jax.ShapeDtypeStruct((n, d), jnp.float32),
  )(x, w, dinv_col)


def _epi1_body(zp_ref, y_ref, dinv_ref, bconv_ref, xn_ref, s_ref):
  i = pl.program_id(0)
  z = zp_ref[0] + zp_ref[1] + y_ref[...]
  xn = z * dinv_ref[...] + bconv_ref[...]
  xn_ref[...] = xn

  @pl.when(i == 0)
  def _():
    s_ref[...] = jnp.zeros_like(s_ref)
  s_ref[...] += jnp.sum(xn, axis=0, keepdims=True)


def _epi2_body(xn_ref, s_ref, wproj_ref, bproj_ref, hmask_ref, gamma_ref,
               beta_ref, x_ref, nval_ref, out_ref):
  xn = xn_ref[...]
  mproj = jnp.dot(s_ref[...] * nval_ref[...],
                  wproj_ref[...], preferred_element_type=jnp.float32)
  mproj = mproj + bproj_ref[...]
  xp = jnp.dot(xn, wproj_ref[...], preferred_element_type=jnp.float32)
  xp = xp + bproj_ref[...]
  p = xp * mproj
  sfull = jnp.dot(p, hmask_ref[...], preferred_element_type=jnp.float32)
  rmax = jnp.max(sfull, axis=1, keepdims=True)
  ex = jnp.exp(sfull - rmax)
  denom = jnp.sum(ex, axis=1, keepdims=True) * (1.0 / 32.0)
  xn = xn * (ex / denom)
  mu = jnp.mean(xn, axis=1, keepdims=True)
  xc = xn - mu
  var = jnp.mean(xc * xc, axis=1, keepdims=True)
  xn = xc * lax.rsqrt(var + 1e-5) * gamma_ref[...] + beta_ref[...]
  out_ref[...] = jnp.maximum(xn, 0.0) + x_ref[...]


def _tc_epilogue(zp, y, dinv_col, b_conv, w_proj, b_proj, hmask, gamma, beta, x):
  n, d = y.shape
  blk = 1000
  xn, s = pl.pallas_call(
      _epi1_body,
      grid=(n // blk,),
      in_specs=[
          pl.BlockSpec((NC, blk, d), lambda i: (0, i, 0)),
          pl.BlockSpec((blk, d), lambda i: (i, 0)),
          pl.BlockSpec((blk, 1), lambda i: (i, 0)),
          pl.BlockSpec((1, d), lambda i: (0, 0)),
      ],
      out_specs=[
          pl.BlockSpec((blk, d), lambda i: (i, 0)),
          pl.BlockSpec((1, d), lambda i: (0, 0)),
      ],
      out_shape=[
          jax.ShapeDtypeStruct((n, d), jnp.float32),
          jax.ShapeDtypeStruct((1, d), jnp.float32),
      ],
  )(zp, y, dinv_col, b_conv)
  nval = jnp.full((1, 1), 1.0 / n, jnp.float32)
  return pl.pallas_call(
      _epi2_body,
      grid=(n // blk,),
      in_specs=[
          pl.BlockSpec((blk, d), lambda i: (i, 0)),
          pl.BlockSpec((1, d), lambda i: (0, 0)),
          pl.BlockSpec((d, d), lambda i: (0, 0)),
          pl.BlockSpec((1, d), lambda i: (0, 0)),
          pl.BlockSpec((d, d), lambda i: (0, 0)),
          pl.BlockSpec((1, d), lambda i: (0, 0)),
          pl.BlockSpec((1, d), lambda i: (0, 0)),
          pl.BlockSpec((blk, d), lambda i: (i, 0)),
          pl.BlockSpec((1, 1), lambda i: (0, 0)),
      ],
      out_specs=pl.BlockSpec((blk, d), lambda i: (i, 0)),
      out_shape=jax.ShapeDtypeStruct((n, d), jnp.float32),
  )(xn, s, w_proj, b_proj, hmask, gamma, beta, x, nval)


def _edge_mm_body(ee_ref, w1_ref, w2_ref, be_ref, a_ref, b_ref):
  ee = ee_ref[...]
  a_ref[...] = jnp.dot(ee, w1_ref[...],
                       preferred_element_type=jnp.float32) + be_ref[...]
  b_ref[...] = jnp.dot(ee, w2_ref[...], preferred_element_type=jnp.float32)


def _tc_edge_mm(ee, w1, w2, be):
  e, d = ee.shape
  blk = 2000
  grid = e // blk
  return pl.pallas_call(
      _edge_mm_body,
      grid=(grid,),
      in_specs=[
          pl.BlockSpec((blk, d), lambda i: (i, 0)),
          pl.BlockSpec((d, d), lambda i: (0, 0)),
          pl.BlockSpec((d, d), lambda i: (0, 0)),
          pl.BlockSpec((1, d), lambda i: (0, 0)),
      ],
      out_specs=[
          pl.BlockSpec((blk, d), lambda i: (i, 0)),
          pl.BlockSpec((blk, d), lambda i: (i, 0)),
      ],
      out_shape=[
          jax.ShapeDtypeStruct((e, d), jnp.float32),
          jax.ShapeDtypeStruct((e, d), jnp.float32),
      ],
  )(ee, w1, w2, be)


@jax.jit
def kernel(x, edge_index, edge_emb, W_conv, b_conv, gamma, beta, W_proj,
           b_proj, W_edge, b_edge):
  n, d = x.shape
  L = W_conv.shape[0]
  h = 4
  hd = d // h

  src = edge_index[0]
  dst = edge_index[1]
  e = src.shape[0]
  ncg = e // 128
  src2 = src.reshape(ncg, 1, 128)
  dst2 = dst.reshape(ncg, 1, 128)

  degp = _sc_degree(dst).reshape(NC, n)
  dinv_row = _tc_dinv(degp)
  dinv_col = dinv_row.reshape(n, 1)

  hmask = jnp.kron(jnp.eye(h, dtype=jnp.float32),
                   jnp.ones((hd, hd), jnp.float32))

  for i in range(L):
    a, b = _tc_edge_mm(edge_emb, W_edge[i][:d], W_edge[i][d:],
                       b_edge[i][None, :])
    y = _tc_prep(x, W_conv[i], dinv_col)
    zp = _sc_node_scatter(y, src2, dst2)
    edge_emb = _sc_edge_combine(a, b, src2, dst2)
    x = _tc_epilogue(zp, y, dinv_col, b_conv[i][None, :], W_proj[i],
                     b_proj[i][None, :], hmask, gamma[i][None, :],
                     beta[i][None, :], x)

  return (x, edge_emb)


# speedup vs baseline: 1.0164x; 1.0164x over previous
import functools

import jax
import jax.numpy as jnp
from jax import lax
from jax.experimental import pallas as pl
from jax.experimental.pallas import tpu as pltpu
from jax.experimental.pallas import tpu_sc as plsc

NC = 2
NS = 16
NW = NC * NS

CH = 80


def _deg_body(dst_hbm, out_hbm, idx_v, ones_v, zero_v, deg_sh, sem):
  n = out_hbm.shape[0] // NC
  e = dst_hbm.shape[0]
  cid = lax.axis_index("c")
  sid = lax.axis_index("s")
  wid = sid * NC + cid
  epw = e // NW
  nch = epw // CH

  for j in range(CH // 16):
    ones_v[pl.ds(j * 16, 16)] = jnp.full((16,), 1.0, jnp.float32)

  @pl.when(sid == 0)
  def _():
    def zfill(j, _):
      zero_v[pl.ds(j * 16, 16)] = jnp.zeros((16,), jnp.float32)
      return 0
    lax.fori_loop(0, n // 16, zfill, 0)
    pltpu.sync_copy(zero_v, deg_sh)

  plsc.subcore_barrier()

  def chunk(c, _):
    base = wid * epw + c * CH
    pltpu.sync_copy(dst_hbm.at[pl.ds(base, CH)], idx_v)
    pltpu.sync_copy(ones_v, deg_sh.at[idx_v], add=True)
    return 0

  lax.fori_loop(0, nch, chunk, 0)
  plsc.subcore_barrier()

  @pl.when(sid == 0)
  def _():
    pltpu.sync_copy(deg_sh, zero_v)
    pltpu.sync_copy(zero_v, out_hbm.at[pl.ds(cid * n, n)])


def _sc_degree(dst):
  e = dst.shape[0]
  n = 10000
  mesh = plsc.VectorSubcoreMesh(core_axis_name="c", subcore_axis_name="s", num_cores=NC, num_subcores=NS)
  return pl.kernel(
      _deg_body,
      out_type=jax.ShapeDtypeStruct((NC * n,), jnp.float32),
      mesh=mesh,
      scratch_types=[
          pltpu.VMEM((CH,), jnp.int32),
          pltpu.VMEM((CH,), jnp.float32),
          pltpu.VMEM((n,), jnp.float32),
          pltpu.VMEM_SHARED((n,), jnp.float32),
          pltpu.SemaphoreType.DMA,
      ],
  )(dst)


def _scatter_body(y_hbm, src_hbm, dst_hbm, out_hbm, is0, is1, id0, id1, rows0,
                  rows1, zero_v, z_sh, sis0, sis1, sid0, sid1, sg0, sg1):
  n, d = y_hbm.shape
  e = src_hbm.shape[0] * src_hbm.shape[1] * src_hbm.shape[2]
  cid = lax.axis_index("c")
  sid = lax.axis_index("s")
  wid = sid * NC + cid
  epw = e // NW
  nch = epw // CH

  zrows = zero_v.shape[0]
  nzg = n // zrows
  nz = (nzg // NS) + jnp.where(sid < (nzg % NS), 1, 0)

  def zfill(j, _):
    for k in range(d // 16):
      zero_v[j, pl.ds(k * 16, 16)] = jnp.zeros((16,), jnp.float32)
    return 0
  lax.fori_loop(0, zrows, zfill, 0)

  def zchunk(k, _):
    g = sid + NS * k
    pltpu.sync_copy(zero_v, z_sh.at[pl.ds(g * zrows, zrows)])
    return 0
  lax.fori_loop(0, nz, zchunk, 0)

  plsc.subcore_barrier()

  idx_s = [is0, is1]
  idx_d = [id0, id1]
  rows = [rows0, rows1]
  sem_is = [sis0, sis1]
  sem_id = [sid0, sid1]
  sem_g = [sg0, sg1]

  def load_idx(c, s):
    pltpu.async_copy(src_hbm.at[wid, c], idx_s[s], sem_is[s])
    pltpu.async_copy(dst_hbm.at[wid, c], idx_d[s], sem_id[s])

  def wait_idx(c, s):
    pltpu.make_async_copy(src_hbm.at[wid, c], idx_s[s], sem_is[s]).wait()
    pltpu.make_async_copy(dst_hbm.at[wid, c], idx_d[s], sem_id[s]).wait()

  load_idx(0, 0)
  wait_idx(0, 0)
  pltpu.async_copy(y_hbm.at[idx_s[0]], rows[0], sem_g[0])
  load_idx(1, 1)

  def chunk(c, _):
    for s in range(2):
      @pl.when(c % 2 == s)
      def _():
        t = 1 - s

        @pl.when(c + 1 < nch)
        def _():
          wait_idx(c + 1, t)
          pltpu.async_copy(y_hbm.at[idx_s[t]], rows[t], sem_g[t])

        pltpu.make_async_copy(y_hbm.at[idx_s[s]], rows[s], sem_g[s]).wait()
        pltpu.sync_copy(rows[s], z_sh.at[idx_d[s]], add=True)

        @pl.when(c + 2 < nch)
        def _():
          load_idx(c + 2, s)
    return 0

  lax.fori_loop(0, nch, chunk, 0)
  plsc.subcore_barrier()

  nog = n // 80
  no = (nog // NS) + jnp.where(sid < (nog % NS), 1, 0)

  def ochunk(k, _):
    g = sid + NS * k
    r0 = g * 80
    pltpu.sync_copy(z_sh.at[pl.ds(r0, 80)], out_hbm.at[cid, pl.ds(r0, 80)])
    return 0
  lax.fori_loop(0, no, ochunk, 0)


def _sc_node_scatter(y, src3, dst3):
  n, d = y.shape
  mesh = plsc.VectorSubcoreMesh(core_axis_name="c", subcore_axis_name="s", num_cores=NC, num_subcores=NS)
  return pl.kernel(
      _scatter_body,
      out_type=jax.ShapeDtypeStruct((NC, n, d), jnp.float32),
      mesh=mesh,
      scratch_types=[
          pltpu.VMEM((CH,), jnp.int32),
          pltpu.VMEM((CH,), jnp.int32),
          pltpu.VMEM((CH,), jnp.int32),
          pltpu.VMEM((CH,), jnp.int32),
          pltpu.VMEM((CH, d), jnp.float32),
          pltpu.VMEM((CH, d), jnp.float32),
          pltpu.VMEM((40, d), jnp.float32),
          pltpu.VMEM_SHARED((n, d), jnp.float32),
          pltpu.SemaphoreType.DMA,
          pltpu.SemaphoreType.DMA,
          pltpu.SemaphoreType.DMA,
          pltpu.SemaphoreType.DMA,
          pltpu.SemaphoreType.DMA,
          pltpu.SemaphoreType.DMA,
      ],
  )(y, src3, dst3)


def _edge_body(a_hbm, b_hbm, src_hbm, dst_hbm, out_hbm, idx_s, idx_d, buf_a0,
               buf_a1, buf_b0, buf_b1, sem_a0, sem_a1, sem_b0, sem_b1,
               sem_o0, sem_o1):
  e, d = out_hbm.shape
  cid = lax.axis_index("c")
  sid = lax.axis_index("s")
  wid = sid * NC + cid
  epw = e // NW
  nch = epw // CH

  pltpu.sync_copy(src_hbm.at[wid], idx_s)
  pltpu.sync_copy(dst_hbm.at[wid], idx_d)

  buf_a = [buf_a0, buf_a1]
  buf_b = [buf_b0, buf_b1]
  sem_a = [sem_a0, sem_a1]
  sem_b = [sem_b0, sem_b1]
  sem_o = [sem_o0, sem_o1]

  pltpu.async_copy(a_hbm.at[idx_s.at[0]], buf_a[0], sem_a[0])
  pltpu.async_copy(b_hbm.at[idx_d.at[0]], buf_b[0], sem_b[0])

  def chunk(c, _):
    for s in range(2):
      @pl.when(c % 2 == s)
      def _():
        t = 1 - s

        @pl.when(c >= 1)
        def _():
          base_p = wid * epw + (c - 1) * CH
          pltpu.make_async_copy(buf_a[t], out_hbm.at[pl.ds(base_p, CH)],
                                sem_o[t]).wait()

        @pl.when(c + 1 < nch)
        def _():
          pltpu.async_copy(a_hbm.at[idx_s.at[c + 1]], buf_a[t], sem_a[t])
          pltpu.async_copy(b_hbm.at[idx_d.at[c + 1]], buf_b[t], sem_b[t])

        pltpu.make_async_copy(a_hbm.at[idx_s.at[c]], buf_a[s], sem_a[s]).wait()
        pltpu.make_async_copy(b_hbm.at[idx_d.at[c]], buf_b[s], sem_b[s]).wait()

        @plsc.parallel_loop(0, CH, 1, unroll=4)
        def add_row(r):
          for j in range(d // 16):
            sl = pl.ds(j * 16, 16)
            buf_a[s][r, sl] = buf_a[s][r, sl] + buf_b[s][r, sl]

        base = wid * epw + c * CH
        pltpu.async_copy(buf_a[s], out_hbm.at[pl.ds(base, CH)], sem_o[s])
    return 0

  lax.fori_loop(0, nch, chunk, 0)

  s = (nch - 1) % 2
  base = wid * epw + (nch - 1) * CH
  pltpu.make_async_copy(buf_a[s], out_hbm.at[pl.ds(base, CH)],
                        sem_o[s]).wait()


def _sc_edge_combine(a, b, src3, dst3):
  e, d = a.shape
  nchw = src3.shape[1]
  mesh = plsc.VectorSubcoreMesh(core_axis_name="c", subcore_axis_name="s", num_cores=NC, num_subcores=NS)
  return pl.kernel(
      _edge_body,
      out_type=jax.ShapeDtypeStruct((e, d), jnp.float32),
      mesh=mesh,
      scratch_types=[
          pltpu.VMEM((nchw, CH), jnp.int32),
          pltpu.VMEM((nchw, CH), jnp.int32),
          pltpu.VMEM((CH, d), jnp.float32),
          pltpu.VMEM((CH, d), jnp.float32),
          pltpu.VMEM((CH, d), jnp.float32),
          pltpu.VMEM((CH, d), jnp.float32),
          pltpu.SemaphoreType.DMA,
          pltpu.SemaphoreType.DMA,
          pltpu.SemaphoreType.DMA,
          pltpu.SemaphoreType.DMA,
          pltpu.SemaphoreType.DMA,
          pltpu.SemaphoreType.DMA,
      ],
  )(a, b, src3, dst3)


def _dinv_body(degp_ref, out_ref):
  deg = degp_ref[0, :] + degp_ref[1, :] + 1.0
  out_ref[0, :] = lax.rsqrt(deg)


def _tc_dinv(degp):
  return pl.pallas_call(
      _dinv_body,
      out_shape=jax.ShapeDtypeStruct((1, degp.shape[1]), jnp.float32),
  )(degp)


def _prep_body(x_ref, w_ref, dinv_ref, y_ref):
  xw = jnp.dot(x_ref[...], w_ref[...], preferred_element_type=jnp.float32)
  y_ref[...] = xw * dinv_ref[...]


def _tc_prep(x, w, dinv_col):
  n, d = x.shape
  return pl.pallas_call(
      _prep_body,
      out_shape=jax.ShapeDtypeStruct((n, d), jnp.float32),
  )(x, w, dinv_col)


def _epi_body(zp_ref, y_ref, dinv_ref, bconv_ref, wproj_ref, bproj_ref,
              hmask_ref, gamma_ref, beta_ref, x_ref, out_ref):
  n, d = y_ref.shape
  z = zp_ref[0] + zp_ref[1] + y_ref[...]
  xn = z * dinv_ref[...] + bconv_ref[...]
  xp = jnp.dot(xn, wproj_ref[...], preferred_element_type=jnp.float32)
  xp = xp + bproj_ref[...]
  m = jnp.mean(xp, axis=0, keepdims=True)
  p = xp * m
  sfull = jnp.dot(p, hmask_ref[...], preferred_element_type=jnp.float32)
  rmax = jnp.max(sfull, axis=1, keepdims=True)
  ex = jnp.exp(sfull - rmax)
  denom = jnp.sum(ex, axis=1, keepdims=True) * (1.0 / 32.0)
  xn = xn * (ex / denom)
  mu = jnp.mean(xn, axis=1, keepdims=True)
  xc = xn - mu
  var = jnp.mean(xc * xc, axis=1, keepdims=True)
  xn = xc * lax.rsqrt(var + 1e-5) * gamma_ref[...] + beta_ref[...]
  out_ref[...] = jnp.maximum(xn, 0.0) + x_ref[...]


def _tc_epilogue(zp, y, dinv_col, b_conv, w_proj, b_proj, hmask, gamma, beta, x):
  n, d = y.shape
  return pl.pallas_call(
      _epi_body,
      out_shape=jax.ShapeDtypeStruct((n, d), jnp.float32),
  )(zp, y, dinv_col, b_conv, w_proj, b_proj, hmask, gamma, beta, x)


def _edge_mm_body(ee_ref, w1_ref, w2_ref, be_ref, a_ref, b_ref):
  ee = ee_ref[...]
  a_ref[...] = jnp.dot(ee, w1_ref[...],
                       preferred_element_type=jnp.float32) + be_ref[...]
  b_ref[...] = jnp.dot(ee, w2_ref[...], preferred_element_type=jnp.float32)


def _tc_edge_mm(ee, w1, w2, be):
  e, d = ee.shape
  blk = 2000
  grid = e // blk
  return pl.pallas_call(
      _edge_mm_body,
      grid=(grid,),
      in_specs=[
          pl.BlockSpec((blk, d), lambda i: (i, 0)),
          pl.BlockSpec((d, d), lambda i: (0, 0)),
          pl.BlockSpec((d, d), lambda i: (0, 0)),
          pl.BlockSpec((1, d), lambda i: (0, 0)),
      ],
      out_specs=[
          pl.BlockSpec((blk, d), lambda i: (i, 0)),
          pl.BlockSpec((blk, d), lambda i: (i, 0)),
      ],
      out_shape=[
          jax.ShapeDtypeStruct((e, d), jnp.float32),
          jax.ShapeDtypeStruct((e, d), jnp.float32),
      ],
  )(ee, w1, w2, be)


@jax.jit
def kernel(x, edge_index, edge_emb, W_conv, b_conv, gamma, beta, W_proj,
           b_proj, W_edge, b_edge):
  n, d = x.shape
  L = W_conv.shape[0]
  h = 4
  hd = d // h

  src = edge_index[0]
  dst = edge_index[1]
  e = src.shape[0]
  nchw = e // (NW * CH)
  src3 = src.reshape(NW, nchw, CH)
  dst3 = dst.reshape(NW, nchw, CH)

  degp = _sc_degree(dst).reshape(NC, n)
  dinv_row = _tc_dinv(degp)
  dinv_col = dinv_row.reshape(n, 1)

  hmask = jnp.kron(jnp.eye(h, dtype=jnp.float32),
                   jnp.ones((hd, hd), jnp.float32))

  for i in range(L):
    y = _tc_prep(x, W_conv[i], dinv_col)
    zp = _sc_node_scatter(y, src3, dst3)
    x = _tc_epilogue(zp, y, dinv_col, b_conv[i][None, :], W_proj[i],
                     b_proj[i][None, :], hmask, gamma[i][None, :],
                     beta[i][None, :], x)
    a, b = _tc_edge_mm(edge_emb, W_edge[i][:d], W_edge[i][d:],
                       b_edge[i][None, :])
    edge_emb = _sc_edge_combine(a, b, src3, dst3)

  return (x, edge_emb)
